# baseline (speedup 1.0000x reference)
import jax
import jax.numpy as jnp
from jax import lax
from jax.experimental import pallas as pl
from jax.experimental.pallas import tpu as pltpu
from jax.experimental.pallas import tpu_sc as plsc

B = 16384
F = 26
K = 16
V = 1000000
NC = 2
NS = 16
NW = NC * NS
BPW = B // NW
CB = 128
CHUNKS = BPW // CB
SB = 16
NBLK = CB // SB

TCOLS = V // 128
VTAIL = V - TCOLS * 128
ROUNDS = TCOLS // NW


def _prep_body(tab_hbm, idx2d_hbm, tail_hbm, tab_out, idx_out,
               tb0, tb1, tb2, tb3, rb0, rb1, rb2, rb3, ibuf, tvec,
               tsem0, tsem1, tsem2, tsem3, wsem0, wsem1, wsem2, wsem3,
               isem, osem):
    tbs = [tb0, tb1, tb2, tb3]
    rbs = [rb0, rb1, rb2, rb3]
    tsems = [tsem0, tsem1, tsem2, tsem3]
    wsems = [wsem0, wsem1, wsem2, wsem3]
    wid = lax.axis_index("s") * NC + lax.axis_index("c")
    wbase = wid * BPW
    lane = lax.iota(jnp.int32, 16)

    for f in range(F):
        pltpu.async_copy(idx2d_hbm.at[f, pl.ds(wbase, BPW)], ibuf.at[f], isem)
    for f in range(F):
        pltpu.make_async_copy(
            idx2d_hbm.at[0, pl.ds(wbase, BPW)], ibuf.at[f], isem
        ).wait()
    for f in range(F):
        pltpu.async_copy(ibuf.at[f], idx_out.at[pl.ds(f * B + wbase, BPW)], osem)

    def tstart(b, tb, sem):
        pltpu.async_copy(
            tab_hbm.at[pl.ds(0, 8), pl.ds(b * 128, 128)], tb.at[pl.ds(0, 8)], sem
        )
        pltpu.async_copy(
            tab_hbm.at[pl.ds(8, 8), pl.ds(b * 128, 128)], tb.at[pl.ds(8, 8)], sem
        )

    def twait(tb, sem):
        for h in range(2):
            pltpu.make_async_copy(
                tab_hbm.at[pl.ds(0, 8), pl.ds(0, 128)], tb.at[pl.ds(h * 8, 8)], sem
            ).wait()

    def transpose(tb, rb):
        G = 8
        prev = None
        for p0 in range(0, 128, G):
            cur = [
                plsc.load_gather(tb, [lane, jnp.full((16,), p0 + i, jnp.int32)])
                for i in range(G)
            ]
            if prev is not None:
                for i in range(G):
                    rb[pl.ds((p0 - G + i) * K, K)] = prev[i]
            prev = cur
        for i in range(G):
            rb[pl.ds((128 - G + i) * K, K)] = prev[i]

    def wout(b, rb, wsem):
        pltpu.async_copy(rb, tab_out.at[pl.ds(b * 2048, 2048)], wsem)

    def wwait(rb, wsem):
        pltpu.make_async_copy(tab_out.at[pl.ds(0, 2048)], rb, wsem).wait()

    NB = 4
    for i in range(NB):
        tstart(i * NW + wid, tbs[i], tsems[i])

    @pl.loop(0, (ROUNDS + NB) // NB)
    def _(g):
        for i in range(NB):
            b = (NB * g + i) * NW + wid

            @pl.when(b < TCOLS)
            def _():
                twait(tbs[i], tsems[i])

                @pl.when(g > 0)
                def _():
                    wwait(rbs[i], wsems[i])

                transpose(tbs[i], rbs[i])
                wout(b, rbs[i], wsems[i])

                @pl.when((NB * g + i + NB) * NW + wid < TCOLS)
                def _():
                    tstart((NB * g + i + NB) * NW + wid, tbs[i], tsems[i])

    for i in range(NB):
        wwait(rbs[i], wsems[i])

    @pl.when(wid == 0)
    def _():
        pltpu.sync_copy(tail_hbm, tvec)
        l64 = lane * VTAIL
        for p0 in range(0, VTAIL, 16):
            vs = [plsc.load_gather(tvec, [l64 + p0 + i]) for i in range(16)]
            for i in range(16):
                rb0[pl.ds((p0 + i) * K, K)] = vs[i]
        pltpu.async_copy(
            rb0.at[pl.ds(0, VTAIL * K)],
            tab_out.at[pl.ds(TCOLS * 128 * K, VTAIL * K)],
            wsem0,
        )
        pltpu.make_async_copy(
            tab_out.at[pl.ds(0, VTAIL * K)], rb0.at[pl.ds(0, VTAIL * K)], wsem0
        ).wait()

    for f in range(F):
        pltpu.make_async_copy(
            idx2d_hbm.at[0, pl.ds(wbase, BPW)], ibuf.at[f], osem
        ).wait()


def _fm_body(idx_hbm, table_hbm, out_hbm, idx_v, buf0, buf1, out_v, isem, sem0, sem1):
    wid = lax.axis_index("s") * NC + lax.axis_index("c")
    wbase = wid * BPW

    for f in range(F):
        pltpu.async_copy(idx_hbm.at[pl.ds(f * B + wbase, BPW)], idx_v.at[f], isem)
    for f in range(F):
        pltpu.make_async_copy(
            idx_hbm.at[pl.ds(wbase, BPW)], idx_v.at[f], isem
        ).wait()

    lane = lax.iota(jnp.int32, 16)

    def start(c, buf, sem):
        for f in range(F):
            pltpu.async_copy(
                table_hbm.at[idx_v.at[f, pl.ds(c * CB, CB)]],
                buf.at[pl.ds(f * CB, CB)],
                sem,
            )

    def wait(buf, sem):
        for f in range(F):
            pltpu.make_async_copy(
                table_hbm.at[idx_v.at[0, pl.ds(0, CB)]],
                buf.at[pl.ds(f * CB, CB)],
                sem,
            ).wait()

    def compute(buf, c):
        @pl.loop(0, NBLK)
        def _(sb):
            base = sb * SB
            s = [None] * SB
            q = [None] * SB
            for f in range(F):
                for be in range(SB):
                    v = buf[f * CB + base + be]
                    if f == 0:
                        s[be] = v
                        q[be] = v * v
                    else:
                        s[be] = s[be] + v
                        q[be] = q[be] + v * v
            acc = jnp.zeros((16,), jnp.float32)
            for be in range(SB):
                r = s[be] * s[be] - q[be]
                acc = jnp.where(lane == be, jnp.sum(r), acc)
            out_v[pl.ds(c * CB + base, SB)] = acc * 0.5

    start(0, buf0, sem0)
    start(1, buf1, sem1)

    @pl.loop(0, CHUNKS // 2)
    def _(g):
        c = g * 2
        wait(buf0, sem0)
        compute(buf0, c)

        @pl.when(c + 2 < CHUNKS)
        def _():
            start(c + 2, buf0, sem0)

        wait(buf1, sem1)
        compute(buf1, c + 1)

        @pl.when(c + 3 < CHUNKS)
        def _():
            start(c + 3, buf1, sem1)

    pltpu.sync_copy(out_v, out_hbm.at[pl.ds(wbase, BPW)])


@jax.jit
def kernel(indices, player_v):
    idx_t2 = jnp.transpose(indices.astype(jnp.int32))
    tab_t = jnp.transpose(player_v)
    tail = tab_t[:, TCOLS * 128:].reshape(-1)
    mesh = plsc.VectorSubcoreMesh(
        core_axis_name="c", subcore_axis_name="s", num_cores=NC, num_subcores=NS
    )
    prep = pl.kernel(
        _prep_body,
        out_type=(
            jax.ShapeDtypeStruct((V * K,), jnp.float32),
            jax.ShapeDtypeStruct((F * B,), jnp.int32),
        ),
        mesh=mesh,
        compiler_params=pltpu.CompilerParams(needs_layout_passes=False),
        scratch_types=[
            pltpu.VMEM((K, 128), jnp.float32),
            pltpu.VMEM((K, 128), jnp.float32),
            pltpu.VMEM((K, 128), jnp.float32),
            pltpu.VMEM((K, 128), jnp.float32),
            pltpu.VMEM((2048,), jnp.float32),
            pltpu.VMEM((2048,), jnp.float32),
            pltpu.VMEM((2048,), jnp.float32),
            pltpu.VMEM((2048,), jnp.float32),
            pltpu.VMEM((F, BPW), jnp.int32),
            pltpu.VMEM((K * VTAIL,), jnp.float32),
            pltpu.SemaphoreType.DMA,
            pltpu.SemaphoreType.DMA,
            pltpu.SemaphoreType.DMA,
            pltpu.SemaphoreType.DMA,
            pltpu.SemaphoreType.DMA,
            pltpu.SemaphoreType.DMA,
            pltpu.SemaphoreType.DMA,
            pltpu.SemaphoreType.DMA,
            pltpu.SemaphoreType.DMA,
            pltpu.SemaphoreType.DMA,
        ],
    )
    table1d, idx1d = prep(tab_t, idx_t2, tail)
    fm = pl.kernel(
        _fm_body,
        out_type=jax.ShapeDtypeStruct((B,), jnp.float32),
        mesh=mesh,
        compiler_params=pltpu.CompilerParams(
            needs_layout_passes=False, use_tc_tiling_on_sc=False
        ),
        scratch_types=[
            pltpu.VMEM((F, BPW), jnp.int32),
            pltpu.VMEM((F * CB, K), jnp.float32),
            pltpu.VMEM((F * CB, K), jnp.float32),
            pltpu.VMEM((BPW,), jnp.float32),
            pltpu.SemaphoreType.DMA,
            pltpu.SemaphoreType.DMA,
            pltpu.SemaphoreType.DMA,
        ],
    )
    return fm(idx1d, jnp.reshape(table1d, (V, K)))


# speedup vs baseline: 1.0036x; 1.0036x over previous
import jax
import jax.numpy as jnp
from jax import lax
from jax.experimental import pallas as pl
from jax.experimental.pallas import tpu as pltpu
from jax.experimental.pallas import tpu_sc as plsc

B = 16384
F = 26
K = 16
V = 1000000
NC = 2
NS = 16
NW = NC * NS
BPW = B // NW
CB = 128
CHUNKS = BPW // CB
SB = 16
NBLK = CB // SB

TCOLS = V // 128
VTAIL = V - TCOLS * 128
ROUNDS = TCOLS // NW


def _prep_body(tab_hbm, idx2d_hbm, tail_hbm, tab_out, idx_out,
               tb0, tb1, rb0, rb1, ibuf, tvec,
               tsem0, tsem1, wsem0, wsem1, isem, osem):
    wid = lax.axis_index("s") * NC + lax.axis_index("c")
    wbase = wid * BPW
    lane = lax.iota(jnp.int32, 16)

    for f in range(F):
        pltpu.async_copy(idx2d_hbm.at[f, pl.ds(wbase, BPW)], ibuf.at[f], isem)
    for f in range(F):
        pltpu.make_async_copy(
            idx2d_hbm.at[0, pl.ds(wbase, BPW)], ibuf.at[f], isem
        ).wait()
    for f in range(F):
        pltpu.async_copy(ibuf.at[f], idx_out.at[pl.ds(f * B + wbase, BPW)], osem)

    def tstart(b, tb, sem):
        pltpu.async_copy(
            tab_hbm.at[pl.ds(0, 8), pl.ds(b * 128, 128)], tb.at[pl.ds(0, 8)], sem
        )
        pltpu.async_copy(
            tab_hbm.at[pl.ds(8, 8), pl.ds(b * 128, 128)], tb.at[pl.ds(8, 8)], sem
        )

    def twait(tb, sem):
        for h in range(2):
            pltpu.make_async_copy(
                tab_hbm.at[pl.ds(0, 8), pl.ds(0, 128)], tb.at[pl.ds(h * 8, 8)], sem
            ).wait()

    def transpose(tb, rb):
        G = 8
        prev = None
        for p0 in range(0, 128, G):
            cur = [
                plsc.load_gather(tb, [lane, jnp.full((16,), p0 + i, jnp.int32)])
                for i in range(G)
            ]
            if prev is not None:
                for i in range(G):
                    rb[pl.ds((p0 - G + i) * K, K)] = prev[i]
            prev = cur
        for i in range(G):
            rb[pl.ds((128 - G + i) * K, K)] = prev[i]

    def wout(b, rb, wsem):
        pltpu.async_copy(rb, tab_out.at[pl.ds(b * 2048, 2048)], wsem)

    def wwait(rb, wsem):
        pltpu.make_async_copy(tab_out.at[pl.ds(0, 2048)], rb, wsem).wait()

    tstart(wid, tb0, tsem0)
    tstart(wid + NW, tb1, tsem1)

    @pl.loop(0, (ROUNDS + 2) // 2)
    def _(g):
        b0 = (2 * g) * NW + wid

        @pl.when(b0 < TCOLS)
        def _():
            twait(tb0, tsem0)

            @pl.when(g > 0)
            def _():
                wwait(rb0, wsem0)

            transpose(tb0, rb0)
            wout(b0, rb0, wsem0)

            @pl.when((2 * g + 2) * NW + wid < TCOLS)
            def _():
                tstart((2 * g + 2) * NW + wid, tb0, tsem0)

        b1 = (2 * g + 1) * NW + wid

        @pl.when(b1 < TCOLS)
        def _():
            twait(tb1, tsem1)

            @pl.when(g > 0)
            def _():
                wwait(rb1, wsem1)

            transpose(tb1, rb1)
            wout(b1, rb1, wsem1)

            @pl.when((2 * g + 3) * NW + wid < TCOLS)
            def _():
                tstart((2 * g + 3) * NW + wid, tb1, tsem1)

    wwait(rb0, wsem0)
    wwait(rb1, wsem1)

    @pl.when(wid == 0)
    def _():
        pltpu.sync_copy(tail_hbm, tvec)
        l64 = lane * VTAIL
        for p0 in range(0, VTAIL, 16):
            vs = [plsc.load_gather(tvec, [l64 + p0 + i]) for i in range(16)]
            for i in range(16):
                rb0[pl.ds((p0 + i) * K, K)] = vs[i]
        pltpu.async_copy(
            rb0.at[pl.ds(0, VTAIL * K)],
            tab_out.at[pl.ds(TCOLS * 128 * K, VTAIL * K)],
            wsem0,
        )
        pltpu.make_async_copy(
            tab_out.at[pl.ds(0, VTAIL * K)], rb0.at[pl.ds(0, VTAIL * K)], wsem0
        ).wait()

    for f in range(F):
        pltpu.make_async_copy(
            idx2d_hbm.at[0, pl.ds(wbase, BPW)], ibuf.at[f], osem
        ).wait()


def _fm_body(idx_hbm, table_hbm, out_hbm, idx_v, buf0, buf1, out_v, isem, sem0, sem1):
    wid = lax.axis_index("s") * NC + lax.axis_index("c")
    wbase = wid * BPW

    for f in range(F):
        pltpu.async_copy(idx_hbm.at[pl.ds(f * B + wbase, BPW)], idx_v.at[f], isem)
    for f in range(F):
        pltpu.make_async_copy(
            idx_hbm.at[pl.ds(wbase, BPW)], idx_v.at[f], isem
        ).wait()

    lane = lax.iota(jnp.int32, 16)

    def start(c, buf, sem):
        for f in range(F):
            pltpu.async_copy(
                table_hbm.at[idx_v.at[f, pl.ds(c * CB, CB)]],
                buf.at[pl.ds(f * CB, CB)],
                sem,
            )

    def wait(buf, sem):
        for f in range(F):
            pltpu.make_async_copy(
                table_hbm.at[idx_v.at[0, pl.ds(0, CB)]],
                buf.at[pl.ds(f * CB, CB)],
                sem,
            ).wait()

    def compute(buf, c):
        @pl.loop(0, NBLK)
        def _(sb):
            base = sb * SB
            s = [None] * SB
            q = [None] * SB
            for f in range(F):
                for be in range(SB):
                    v = buf[f * CB + base + be]
                    if f == 0:
                        s[be] = v
                        q[be] = v * v
                    else:
                        s[be] = s[be] + v
                        q[be] = q[be] + v * v
            acc = jnp.zeros((16,), jnp.float32)
            for be in range(SB):
                r = s[be] * s[be] - q[be]
                acc = jnp.where(lane == be, jnp.sum(r), acc)
            out_v[pl.ds(c * CB + base, SB)] = acc * 0.5

    start(0, buf0, sem0)
    start(1, buf1, sem1)

    @pl.loop(0, CHUNKS // 2)
    def _(g):
        c = g * 2
        wait(buf0, sem0)
        compute(buf0, c)

        @pl.when(c + 2 < CHUNKS)
        def _():
            start(c + 2, buf0, sem0)

        wait(buf1, sem1)
        compute(buf1, c + 1)

        @pl.when(c + 3 < CHUNKS)
        def _():
            start(c + 3, buf1, sem1)

    pltpu.sync_copy(out_v, out_hbm.at[pl.ds(wbase, BPW)])


@jax.jit
def kernel(indices, player_v):
    idx_t2 = jnp.transpose(indices.astype(jnp.int32))
    tab_t = jnp.transpose(player_v)
    tail = tab_t[:, TCOLS * 128:].reshape(-1)
    mesh = plsc.VectorSubcoreMesh(
        core_axis_name="c", subcore_axis_name="s", num_cores=NC, num_subcores=NS
    )
    prep = pl.kernel(
        _prep_body,
        out_type=(
            jax.ShapeDtypeStruct((V * K,), jnp.float32),
            jax.ShapeDtypeStruct((F * B,), jnp.int32),
        ),
        mesh=mesh,
        compiler_params=pltpu.CompilerParams(needs_layout_passes=False),
        scratch_types=[
            pltpu.VMEM((K, 128), jnp.float32),
            pltpu.VMEM((K, 128), jnp.float32),
            pltpu.VMEM((2048,), jnp.float32),
            pltpu.VMEM((2048,), jnp.float32),
            pltpu.VMEM((F, BPW), jnp.int32),
            pltpu.VMEM((K * VTAIL,), jnp.float32),
            pltpu.SemaphoreType.DMA,
            pltpu.SemaphoreType.DMA,
            pltpu.SemaphoreType.DMA,
            pltpu.SemaphoreType.DMA,
            pltpu.SemaphoreType.DMA,
            pltpu.SemaphoreType.DMA,
        ],
    )
    table1d, idx1d = prep(tab_t, idx_t2, tail)
    fm = pl.kernel(
        _fm_body,
        out_type=jax.ShapeDtypeStruct((B,), jnp.float32),
        mesh=mesh,
        compiler_params=pltpu.CompilerParams(
            needs_layout_passes=False, use_tc_tiling_on_sc=False
        ),
        scratch_types=[
            pltpu.VMEM((F, BPW), jnp.int32),
            pltpu.VMEM((F * CB, K), jnp.float32),
            pltpu.VMEM((F * CB, K), jnp.float32),
            pltpu.VMEM((BPW,), jnp.float32),
            pltpu.SemaphoreType.DMA,
            pltpu.SemaphoreType.DMA,
            pltpu.SemaphoreType.DMA,
        ],
    )
    return fm(idx1d, jnp.reshape(table1d, (V, K)))
